# Initial kernel scaffold; baseline (speedup 1.0000x reference)
import jax
import jax.numpy as jnp
from jax.experimental import pallas as pl


def kernel(xyz, features, fc1_w, fc1_b, fc2_w, fc2_b, d1_w, d1_b, d2_w, d2_b, g1_w, g1_b, g2_w, g2_b, wq, wk, wv):
    raise NotImplementedError("write your pallas kernel here")


# speedup vs baseline: 16.3130x; 16.3130x over previous
import functools

import jax
import jax.numpy as jnp
import numpy as np
from jax import lax
from jax.experimental import pallas as pl
from jax.experimental.pallas import tpu as pltpu
from jax.experimental.pallas import tpu_sc as plsc

B, N, DP, DM, K = 4, 4096, 32, 64, 16
XP = 8
RB = 128
TB = 256
TOT = B * N * K

SC_NC, SC_NS = 2, 16
SC_NW = SC_NC * SC_NS
PW = TOT // SC_NW
CH = 128


def _knn_body(xr_ref, xa_ref, out_ref):
    b = pl.program_id(0)
    xr = xr_ref[0]
    xa = xa_ref[0]
    rn = jnp.sum(xr * xr, axis=-1)
    cn = jnp.sum(xa * xa, axis=-1)
    ab = lax.dot_general(xr, xa, (((1,), (1,)), ((), ())),
                         preferred_element_type=jnp.float32)
    d = (rn[:, None] + cn[None, :]) - 2.0 * ab
    col = lax.broadcasted_iota(jnp.int32, (RB, N), 1)
    kcol = lax.broadcasted_iota(jnp.int32, (RB, K), 1)
    acc = jnp.zeros((RB, K), jnp.int32)
    base = b * N
    for j in range(K):
        m = jnp.min(d, axis=1)
        idx = jnp.min(jnp.where(d == m[:, None], col, N), axis=1)
        acc = jnp.where(kcol == j, (idx + base)[:, None], acc)
        d = jnp.where(col == idx[:, None], jnp.inf, d)
    out_ref[0] = acc


def _knn_indices(xyz_pad):
    return pl.pallas_call(
        _knn_body,
        grid=(B, N // RB),
        in_specs=[
            pl.BlockSpec((1, RB, XP), lambda b, r: (b, r, 0)),
            pl.BlockSpec((1, N, XP), lambda b, r: (b, 0, 0)),
        ],
        out_specs=pl.BlockSpec((1, RB, K), lambda b, r: (b, r, 0)),
        out_shape=jax.ShapeDtypeStruct((B, N, K), jnp.int32),
    )(xyz_pad, xyz_pad)


@functools.cache
def _sc_gather_kernel():
    mesh = plsc.VectorSubcoreMesh(core_axis_name="c", subcore_axis_name="s")

    @functools.partial(
        pl.kernel,
        mesh=mesh,
        out_type=[
            jax.ShapeDtypeStruct((TOT, DP), jnp.float32),
            jax.ShapeDtypeStruct((TOT, XP), jnp.float32),
        ],
        scratch_types=[
            pltpu.VMEM((CH,), jnp.int32),
            pltpu.VMEM((CH, DP), jnp.float32),
            pltpu.VMEM((CH, XP), jnp.float32),
            pltpu.SemaphoreType.DMA,
            pltpu.SemaphoreType.DMA,
        ],
        compiler_params=pltpu.CompilerParams(use_tc_tiling_on_sc=False),
    )
    def gather(feat_hbm, xyzp_hbm, idx_hbm, featg_hbm, xyzg_hbm,
               idx_v, rows_v, xyzr_v, sem_f, sem_x):
        wid = lax.axis_index("s") * SC_NC + lax.axis_index("c")
        base = wid * PW

        def body(i, carry):
            off = base + i * CH
            pltpu.sync_copy(idx_hbm.at[pl.ds(off, CH)], idx_v)
            cf = pltpu.async_copy(feat_hbm.at[idx_v], rows_v, sem_f)
            cx = pltpu.async_copy(xyzp_hbm.at[idx_v], xyzr_v, sem_x)
            cf.wait()
            cx.wait()
            pltpu.sync_copy(rows_v, featg_hbm.at[pl.ds(off, CH)])
            pltpu.sync_copy(xyzr_v, xyzg_hbm.at[pl.ds(off, CH)])
            return carry

        lax.fori_loop(0, PW // CH, body, 0)

    return gather


def _sc_gather(feat2, xyzp2, idxf):
    return _sc_gather_kernel()(feat2, xyzp2, idxf)


def _attn_body(feat_ref, featg_ref, xyzg_ref, xyzq_ref,
               fc1w_ref, fc1b_ref, fc2w_ref, fc2b_ref,
               d1w_ref, d1b_ref, d2w_ref, d2b_ref,
               g1w_ref, g1b_ref, g2w_ref, g2b_ref,
               wq_ref, wk_ref, wv_ref,
               res_ref, attn_ref):
    featb = feat_ref[0]
    featg = featg_ref[0]
    xyzg = xyzg_ref[0]
    xyzq = xyzq_ref[0]

    fc1w = fc1w_ref[...]
    fc1b = fc1b_ref[...]

    def mm(a, w):
        return jnp.dot(a, w, preferred_element_type=jnp.float32)

    xb = mm(featb, fc1w) + fc1b
    q = mm(xb, wq_ref[...])
    xg = mm(featg, fc1w) + fc1b
    k_ = mm(xg, wk_ref[...])
    v_ = mm(xg, wv_ref[...])

    pos = xyzq - xyzg
    pe = mm(jax.nn.relu(mm(pos, d1w_ref[...]) + d1b_ref[...]),
            d2w_ref[...]) + d2b_ref[...]

    kp = pe - k_
    a3 = q[:, None, :] + kp.reshape(TB, K, DM)
    h = jax.nn.relu(mm(a3.reshape(TB * K, DM), g1w_ref[...]) + g1b_ref[...])
    logits = (mm(h, g2w_ref[...]) + g2b_ref[...]) * (1.0 / np.sqrt(DM))
    l3 = logits.reshape(TB, K, DM)
    mx = jnp.max(l3, axis=1, keepdims=True)
    e = jnp.exp(l3 - mx)
    attn = e / jnp.sum(e, axis=1, keepdims=True)

    vp = (v_ + pe).reshape(TB, K, DM)
    red = jnp.sum(attn * vp, axis=1)
    res_ref[0] = mm(red, fc2w_ref[...]) + fc2b_ref[...] + featb
    attn_ref[0] = attn.reshape(TB * K, DM)


def _attn_stage(features, featg, xyzg, xyzq, weights):
    (fc1w, fc1b, fc2w, fc2b, d1w, d1b, d2w, d2b,
     g1w, g1b, g2w, g2b, wq, wk, wv) = weights

    def wspec(arr):
        nd = arr.ndim
        return pl.BlockSpec(arr.shape, lambda b, t: (0,) * nd)

    d1w = jnp.pad(d1w, ((0, XP - 3), (0, 0)))
    wlist = [fc1w, fc1b, fc2w, fc2b, d1w, d1b, d2w, d2b,
             g1w, g1b, g2w, g2b, wq, wk, wv]
    wlist = [w.reshape(1, -1) if w.ndim == 1 else w for w in wlist]
    return pl.pallas_call(
        _attn_body,
        grid=(B, N // TB),
        in_specs=[
            pl.BlockSpec((1, TB, DP), lambda b, t: (b, t, 0)),
            pl.BlockSpec((1, TB * K, DP), lambda b, t: (b, t, 0)),
            pl.BlockSpec((1, TB * K, XP), lambda b, t: (b, t, 0)),
            pl.BlockSpec((1, TB * K, XP), lambda b, t: (b, t, 0)),
        ] + [wspec(w) for w in wlist],
        out_specs=[
            pl.BlockSpec((1, TB, DP), lambda b, t: (b, t, 0)),
            pl.BlockSpec((1, TB * K, DM), lambda b, t: (b, t, 0)),
        ],
        out_shape=[
            jax.ShapeDtypeStruct((B, N, DP), jnp.float32),
            jax.ShapeDtypeStruct((B, N * K, DM), jnp.float32),
        ],
    )(features, featg, xyzg, xyzq, *wlist)


def kernel(xyz, features, fc1_w, fc1_b, fc2_w, fc2_b, d1_w, d1_b, d2_w, d2_b,
           g1_w, g1_b, g2_w, g2_b, wq, wk, wv):
    xyz_pad = jnp.pad(xyz, ((0, 0), (0, 0), (0, XP - 3)))
    knn = _knn_indices(xyz_pad)
    featg, xyzg = _sc_gather(
        features.reshape(B * N, DP),
        xyz_pad.reshape(B * N, XP),
        knn.reshape(TOT),
    )
    xyzq = jnp.broadcast_to(
        xyz_pad[:, :, None, :], (B, N, K, XP)).reshape(B, N * K, XP)
    weights = (fc1_w, fc1_b, fc2_w, fc2_b, d1_w, d1_b, d2_w, d2_b,
               g1_w, g1_b, g2_w, g2_b, wq, wk, wv)
    res, attn2 = _attn_stage(
        features,
        featg.reshape(B, N * K, DP),
        xyzg.reshape(B, N * K, XP),
        xyzq,
        weights,
    )
    return res, attn2.reshape(B, N, K, DM)


# speedup vs baseline: 20.4849x; 1.2557x over previous
import functools

import jax
import jax.numpy as jnp
import numpy as np
from jax import lax
from jax.experimental import pallas as pl
from jax.experimental.pallas import tpu as pltpu
from jax.experimental.pallas import tpu_sc as plsc

B, N, DP, DM, K = 4, 4096, 32, 64, 16
XP = 8
RB = 256
TB = 256
TOT = B * N * K

SC_NC, SC_NS = 2, 16
SC_NW = SC_NC * SC_NS
PW = TOT // SC_NW
CH = 128


def _knn_body(xr_ref, xa_ref, out_ref):
    b = pl.program_id(0)
    xr = xr_ref[0]
    xa = xa_ref[0]
    rn = jnp.sum(xr * xr, axis=-1)
    cn = jnp.sum(xa * xa, axis=-1)
    ab = lax.dot_general(xr, xa, (((1,), (1,)), ((), ())),
                         preferred_element_type=jnp.float32)
    d = (rn[:, None] + cn[None, :]) - 2.0 * ab
    col = lax.broadcasted_iota(jnp.int32, (RB, N), 1).astype(jnp.float32)
    kcol = lax.broadcasted_iota(jnp.int32, (RB, K), 1)
    acc = jnp.zeros((RB, K), jnp.float32)
    for j in range(K):
        m = jnp.min(d, axis=1)
        idx = jnp.min(jnp.where(d == m[:, None], col, float(N)), axis=1)
        acc = jnp.where(kcol == j, idx[:, None], acc)
        d = jnp.where(col == idx[:, None], jnp.inf, d)
    out_ref[0] = acc.astype(jnp.int32) + b * N


def _knn_indices(xyz_pad):
    return pl.pallas_call(
        _knn_body,
        grid=(B, N // RB),
        in_specs=[
            pl.BlockSpec((1, RB, XP), lambda b, r: (b, r, 0)),
            pl.BlockSpec((1, N, XP), lambda b, r: (b, 0, 0)),
        ],
        out_specs=pl.BlockSpec((1, RB, K), lambda b, r: (b, r, 0)),
        out_shape=jax.ShapeDtypeStruct((B, N, K), jnp.int32),
    )(xyz_pad, xyz_pad)


@functools.cache
def _sc_gather_kernel():
    mesh = plsc.VectorSubcoreMesh(core_axis_name="c", subcore_axis_name="s")

    @functools.partial(
        pl.kernel,
        mesh=mesh,
        out_type=[
            jax.ShapeDtypeStruct((TOT, DP), jnp.float32),
            jax.ShapeDtypeStruct((TOT, XP), jnp.float32),
        ],
        scratch_types=[
            pltpu.VMEM((CH,), jnp.int32),
            pltpu.VMEM((CH, DP), jnp.float32),
            pltpu.VMEM((CH, XP), jnp.float32),
            pltpu.SemaphoreType.DMA,
            pltpu.SemaphoreType.DMA,
        ],
        compiler_params=pltpu.CompilerParams(use_tc_tiling_on_sc=False),
    )
    def gather(feat_hbm, xyzp_hbm, idx_hbm, featg_hbm, xyzg_hbm,
               idx_v, rows_v, xyzr_v, sem_f, sem_x):
        wid = lax.axis_index("s") * SC_NC + lax.axis_index("c")
        base = wid * PW

        def body(i, carry):
            off = base + i * CH
            pltpu.sync_copy(idx_hbm.at[pl.ds(off, CH)], idx_v)
            cf = pltpu.async_copy(feat_hbm.at[idx_v], rows_v, sem_f)
            cx = pltpu.async_copy(xyzp_hbm.at[idx_v], xyzr_v, sem_x)
            cf.wait()
            cx.wait()
            pltpu.sync_copy(rows_v, featg_hbm.at[pl.ds(off, CH)])
            pltpu.sync_copy(xyzr_v, xyzg_hbm.at[pl.ds(off, CH)])
            return carry

        lax.fori_loop(0, PW // CH, body, 0)

    return gather


def _sc_gather(feat2, xyzp2, idxf):
    return _sc_gather_kernel()(feat2, xyzp2, idxf)


def _attn_body(feat_ref, featg_ref, xyzg_ref, xyzq_ref,
               fc1w_ref, fc1b_ref, fc2w_ref, fc2b_ref,
               d1w_ref, d1b_ref, d2w_ref, d2b_ref,
               g1w_ref, g1b_ref, g2w_ref, g2b_ref,
               wq_ref, wk_ref, wv_ref,
               res_ref, attn_ref):
    featb = feat_ref[0]
    featg = featg_ref[0]
    xyzg = xyzg_ref[0]
    xyzq = xyzq_ref[0]

    fc1w = fc1w_ref[...]
    fc1b = fc1b_ref[...]

    def mm(a, w):
        return jnp.dot(a, w, preferred_element_type=jnp.float32)

    xb = mm(featb, fc1w) + fc1b
    q = mm(xb, wq_ref[...])
    xg = mm(featg, fc1w) + fc1b
    k_ = mm(xg, wk_ref[...])
    v_ = mm(xg, wv_ref[...])

    pos = xyzq - xyzg
    pe = mm(jax.nn.relu(mm(pos, d1w_ref[...]) + d1b_ref[...]),
            d2w_ref[...]) + d2b_ref[...]

    kp = pe - k_
    a3 = q[:, None, :] + kp.reshape(TB, K, DM)
    h = jax.nn.relu(mm(a3.reshape(TB * K, DM), g1w_ref[...]) + g1b_ref[...])
    logits = (mm(h, g2w_ref[...]) + g2b_ref[...]) * (1.0 / np.sqrt(DM))
    l3 = logits.reshape(TB, K, DM)
    mx = jnp.max(l3, axis=1, keepdims=True)
    e = jnp.exp(l3 - mx)
    attn = e / jnp.sum(e, axis=1, keepdims=True)

    vp = (v_ + pe).reshape(TB, K, DM)
    red = jnp.sum(attn * vp, axis=1)
    res_ref[0] = mm(red, fc2w_ref[...]) + fc2b_ref[...] + featb
    attn_ref[0] = attn.reshape(TB * K, DM)


def _attn_stage(features, featg, xyzg, xyzq, weights):
    (fc1w, fc1b, fc2w, fc2b, d1w, d1b, d2w, d2b,
     g1w, g1b, g2w, g2b, wq, wk, wv) = weights

    def wspec(arr):
        nd = arr.ndim
        return pl.BlockSpec(arr.shape, lambda b, t: (0,) * nd)

    d1w = jnp.pad(d1w, ((0, XP - 3), (0, 0)))
    wlist = [fc1w, fc1b, fc2w, fc2b, d1w, d1b, d2w, d2b,
             g1w, g1b, g2w, g2b, wq, wk, wv]
    wlist = [w.reshape(1, -1) if w.ndim == 1 else w for w in wlist]
    return pl.pallas_call(
        _attn_body,
        grid=(B, N // TB),
        in_specs=[
            pl.BlockSpec((1, TB, DP), lambda b, t: (b, t, 0)),
            pl.BlockSpec((1, TB * K, DP), lambda b, t: (b, t, 0)),
            pl.BlockSpec((1, TB * K, XP), lambda b, t: (b, t, 0)),
            pl.BlockSpec((1, TB * K, XP), lambda b, t: (b, t, 0)),
        ] + [wspec(w) for w in wlist],
        out_specs=[
            pl.BlockSpec((1, TB, DP), lambda b, t: (b, t, 0)),
            pl.BlockSpec((1, TB * K, DM), lambda b, t: (b, t, 0)),
        ],
        out_shape=[
            jax.ShapeDtypeStruct((B, N, DP), jnp.float32),
            jax.ShapeDtypeStruct((B, N * K, DM), jnp.float32),
        ],
    )(features, featg, xyzg, xyzq, *wlist)


def kernel(xyz, features, fc1_w, fc1_b, fc2_w, fc2_b, d1_w, d1_b, d2_w, d2_b,
           g1_w, g1_b, g2_w, g2_b, wq, wk, wv):
    xyz_pad = jnp.pad(xyz, ((0, 0), (0, 0), (0, XP - 3)))
    knn = _knn_indices(xyz_pad)
    featg, xyzg = _sc_gather(
        features.reshape(B * N, DP),
        xyz_pad.reshape(B * N, XP),
        knn.reshape(TOT),
    )
    xyzq = jnp.broadcast_to(
        xyz_pad[:, :, None, :], (B, N, K, XP)).reshape(B, N * K, XP)
    weights = (fc1_w, fc1_b, fc2_w, fc2_b, d1_w, d1_b, d2_w, d2_b,
               g1_w, g1_b, g2_w, g2_b, wq, wk, wv)
    res, attn2 = _attn_stage(
        features,
        featg.reshape(B, N * K, DP),
        xyzg.reshape(B, N * K, XP),
        xyzq,
        weights,
    )
    return res, attn2.reshape(B, N, K, DM)


# speedup vs baseline: 22.2905x; 1.0881x over previous
import functools

import jax
import jax.numpy as jnp
import numpy as np
from jax import lax
from jax.experimental import pallas as pl
from jax.experimental.pallas import tpu as pltpu
from jax.experimental.pallas import tpu_sc as plsc

B, N, DP, DM, K = 4, 4096, 32, 64, 16
XP = 8
RB = 256
TB = 256
TOT = B * N * K

SC_NC, SC_NS = 2, 16
SC_NW = SC_NC * SC_NS
PW = TOT // SC_NW
CH = 128


def _knn_body(xr_ref, xa_ref, out_ref):
    b = pl.program_id(0)
    xr = xr_ref[0]
    xa = xa_ref[0]
    rn = jnp.sum(xr * xr, axis=-1)
    cn = jnp.sum(xa * xa, axis=-1)
    ab = lax.dot_general(xr, xa, (((1,), (1,)), ((), ())),
                         preferred_element_type=jnp.float32)
    d = (rn[:, None] + cn[None, :]) - 2.0 * ab
    col = lax.broadcasted_iota(jnp.int32, (RB, N), 1).astype(jnp.float32)
    kcol = lax.broadcasted_iota(jnp.int32, (RB, K), 1)
    acc = jnp.zeros((RB, K), jnp.float32)
    for j in range(K):
        m = jnp.min(d, axis=1)
        idx = jnp.min(jnp.where(d == m[:, None], col, float(N)), axis=1)
        acc = jnp.where(kcol == j, idx[:, None], acc)
        d = jnp.where(col == idx[:, None], jnp.inf, d)
    out_ref[0] = acc.astype(jnp.int32) + b * N


def _knn_indices(xyz_pad):
    return pl.pallas_call(
        _knn_body,
        grid=(B, N // RB),
        in_specs=[
            pl.BlockSpec((1, RB, XP), lambda b, r: (b, r, 0)),
            pl.BlockSpec((1, N, XP), lambda b, r: (b, 0, 0)),
        ],
        out_specs=pl.BlockSpec((1, RB, K), lambda b, r: (b, r, 0)),
        out_shape=jax.ShapeDtypeStruct((B, N, K), jnp.int32),
    )(xyz_pad, xyz_pad)


GRP = 8
GW = GRP * CH


@functools.cache
def _sc_gather_kernel():
    mesh = plsc.VectorSubcoreMesh(core_axis_name="c", subcore_axis_name="s")

    @functools.partial(
        pl.kernel,
        mesh=mesh,
        out_type=[
            jax.ShapeDtypeStruct((TOT, DP), jnp.float32),
            jax.ShapeDtypeStruct((TOT, XP), jnp.float32),
        ],
        scratch_types=[
            pltpu.VMEM((PW,), jnp.int32),
            pltpu.VMEM((GW, DP), jnp.float32),
            pltpu.VMEM((GW, XP), jnp.float32),
            pltpu.SemaphoreType.DMA,
            pltpu.SemaphoreType.DMA,
        ],
        compiler_params=pltpu.CompilerParams(use_tc_tiling_on_sc=False),
    )
    def gather(feat_hbm, xyzp_hbm, idx_hbm, featg_hbm, xyzg_hbm,
               idx_v, rows_v, xyzr_v, sem_f, sem_x):
        wid = lax.axis_index("s") * SC_NC + lax.axis_index("c")
        base = wid * PW
        pltpu.sync_copy(idx_hbm.at[pl.ds(base, PW)], idx_v)

        def body(g, carry):
            for bi in range(GRP):
                isl = idx_v.at[pl.ds(g * GW + bi * CH, CH)]
                pltpu.async_copy(feat_hbm.at[isl],
                                 rows_v.at[pl.ds(bi * CH, CH)], sem_f)
                pltpu.async_copy(xyzp_hbm.at[isl],
                                 xyzr_v.at[pl.ds(bi * CH, CH)], sem_x)
            pltpu.make_async_copy(feat_hbm.at[pl.ds(0, GW)], rows_v,
                                  sem_f).wait()
            pltpu.make_async_copy(xyzp_hbm.at[pl.ds(0, GW)], xyzr_v,
                                  sem_x).wait()
            off = base + g * GW
            pltpu.sync_copy(rows_v, featg_hbm.at[pl.ds(off, GW)])
            pltpu.sync_copy(xyzr_v, xyzg_hbm.at[pl.ds(off, GW)])
            return carry

        lax.fori_loop(0, PW // GW, body, 0)

    return gather


def _sc_gather(feat2, xyzp2, idxf):
    return _sc_gather_kernel()(feat2, xyzp2, idxf)


def _attn_body(feat_ref, featg_ref, xyzg_ref, xyzb_ref,
               fc1w_ref, fc1b_ref, fc2w_ref, fc2b_ref,
               d1w_ref, d1b_ref, d2w_ref, d2b_ref,
               g1w_ref, g1b_ref, g2w_ref, g2b_ref,
               wq_ref, wk_ref, wv_ref,
               res_ref, attn_ref):
    featb = feat_ref[0]
    featg = featg_ref[0]
    xyzg = xyzg_ref[0]
    xyzb = xyzb_ref[0]
    xyzq = (xyzb[:, None, :] + jnp.zeros((TB, K, XP), jnp.float32)
            ).reshape(TB * K, XP)

    fc1w = fc1w_ref[...]
    fc1b = fc1b_ref[...]

    def mm(a, w):
        return jnp.dot(a, w, preferred_element_type=jnp.float32)

    xb = mm(featb, fc1w) + fc1b
    q = mm(xb, wq_ref[...])
    xg = mm(featg, fc1w) + fc1b
    k_ = mm(xg, wk_ref[...])
    v_ = mm(xg, wv_ref[...])

    pos = xyzq - xyzg
    pe = mm(jax.nn.relu(mm(pos, d1w_ref[...]) + d1b_ref[...]),
            d2w_ref[...]) + d2b_ref[...]

    kp = pe - k_
    a3 = q[:, None, :] + kp.reshape(TB, K, DM)
    h = jax.nn.relu(mm(a3.reshape(TB * K, DM), g1w_ref[...]) + g1b_ref[...])
    logits = (mm(h, g2w_ref[...]) + g2b_ref[...]) * (1.0 / np.sqrt(DM))
    l3 = logits.reshape(TB, K, DM)
    mx = jnp.max(l3, axis=1, keepdims=True)
    e = jnp.exp(l3 - mx)
    attn = e / jnp.sum(e, axis=1, keepdims=True)

    vp = (v_ + pe).reshape(TB, K, DM)
    red = jnp.sum(attn * vp, axis=1)
    res_ref[0] = mm(red, fc2w_ref[...]) + fc2b_ref[...] + featb
    attn_ref[0] = attn.reshape(TB * K, DM)


def _attn_stage(features, featg, xyzg, xyz_pad, weights):
    (fc1w, fc1b, fc2w, fc2b, d1w, d1b, d2w, d2b,
     g1w, g1b, g2w, g2b, wq, wk, wv) = weights

    def wspec(arr):
        nd = arr.ndim
        return pl.BlockSpec(arr.shape, lambda b, t: (0,) * nd)

    d1w = jnp.pad(d1w, ((0, XP - 3), (0, 0)))
    wlist = [fc1w, fc1b, fc2w, fc2b, d1w, d1b, d2w, d2b,
             g1w, g1b, g2w, g2b, wq, wk, wv]
    wlist = [w.reshape(1, -1) if w.ndim == 1 else w for w in wlist]
    return pl.pallas_call(
        _attn_body,
        grid=(B, N // TB),
        in_specs=[
            pl.BlockSpec((1, TB, DP), lambda b, t: (b, t, 0)),
            pl.BlockSpec((1, TB * K, DP), lambda b, t: (b, t, 0)),
            pl.BlockSpec((1, TB * K, XP), lambda b, t: (b, t, 0)),
            pl.BlockSpec((1, TB, XP), lambda b, t: (b, t, 0)),
        ] + [wspec(w) for w in wlist],
        out_specs=[
            pl.BlockSpec((1, TB, DP), lambda b, t: (b, t, 0)),
            pl.BlockSpec((1, TB * K, DM), lambda b, t: (b, t, 0)),
        ],
        out_shape=[
            jax.ShapeDtypeStruct((B, N, DP), jnp.float32),
            jax.ShapeDtypeStruct((B, N * K, DM), jnp.float32),
        ],
    )(features, featg, xyzg, xyz_pad, *wlist)


def kernel(xyz, features, fc1_w, fc1_b, fc2_w, fc2_b, d1_w, d1_b, d2_w, d2_b,
           g1_w, g1_b, g2_w, g2_b, wq, wk, wv):
    xyz_pad = jnp.pad(xyz, ((0, 0), (0, 0), (0, XP - 3)))
    knn = _knn_indices(xyz_pad)
    featg, xyzg = _sc_gather(
        features.reshape(B * N, DP),
        xyz_pad.reshape(B * N, XP),
        knn.reshape(TOT),
    )
    weights = (fc1_w, fc1_b, fc2_w, fc2_b, d1_w, d1_b, d2_w, d2_b,
               g1_w, g1_b, g2_w, g2_b, wq, wk, wv)
    res, attn2 = _attn_stage(
        features,
        featg.reshape(B, N * K, DP),
        xyzg.reshape(B, N * K, XP),
        xyz_pad,
        weights,
    )
    return res, attn2.reshape(B, N, K, DM)
